# baseline (speedup 1.0000x reference)
import functools

import jax
import jax.numpy as jnp
from jax import lax
from jax.experimental import pallas as pl
from jax.experimental.pallas import tpu as pltpu
from jax.experimental.pallas import tpu_sc as plsc

D = 64
IDX_CHUNK = 128


def _make_sc_gather(n_users, n_items, B):
    info = plsc.get_sparse_core_info()
    NC, NS = info.num_cores, info.num_subcores
    NW = NC * NS
    assert B % (NW * IDX_CHUNK) == 0
    n_chunks = B // (NW * IDX_CHUNK)

    mesh = plsc.VectorSubcoreMesh(core_axis_name="c", subcore_axis_name="s")

    @functools.partial(
        pl.kernel,
        mesh=mesh,
        out_type=(
            jax.ShapeDtypeStruct((NW, n_chunks, IDX_CHUNK, D), jnp.float32),
            jax.ShapeDtypeStruct((NW, n_chunks, IDX_CHUNK, D), jnp.float32),
        ),
        scratch_types=[
            pltpu.VMEM((n_chunks, IDX_CHUNK), jnp.int32),
            pltpu.VMEM((n_chunks, IDX_CHUNK), jnp.int32),
            pltpu.VMEM((n_chunks, IDX_CHUNK, D), jnp.float32),
            pltpu.VMEM((n_chunks, IDX_CHUNK, D), jnp.float32),
            pltpu.SemaphoreType.DMA,
        ],
        compiler_params=pltpu.CompilerParams(use_tc_tiling_on_sc=False),
    )
    def sc_gather(uidx_hbm, iidx_hbm, utab_hbm, itab_hbm, uout_hbm, iout_hbm,
                  uidx_v, iidx_v, urows_v, irows_v, sem):
        wid = lax.axis_index("s") * NC + lax.axis_index("c")
        pltpu.sync_copy(uidx_hbm.at[wid], uidx_v)
        pltpu.sync_copy(iidx_hbm.at[wid], iidx_v)
        copies = []
        for j in range(n_chunks):
            copies.append(
                pltpu.async_copy(utab_hbm.at[uidx_v.at[j]], urows_v.at[j], sem))
            copies.append(
                pltpu.async_copy(itab_hbm.at[iidx_v.at[j]], irows_v.at[j], sem))
        for c in copies:
            c.wait()
        pltpu.sync_copy(urows_v, uout_hbm.at[wid])
        pltpu.sync_copy(irows_v, iout_hbm.at[wid])

    return sc_gather, NW, n_chunks


def _mlp_body(u_ref, i_ref, w1_ref, b1_ref, w2_ref, b2_ref, o_ref):
    u = u_ref[...]
    it = i_ref[...]
    e = u * it
    x = jnp.concatenate([e, u, it], axis=1)
    h = jnp.maximum(
        jnp.dot(x, w1_ref[...], preferred_element_type=jnp.float32)
        + b1_ref[...], 0.0)
    logits = jnp.dot(h, w2_ref[...], preferred_element_type=jnp.float32) \
        + b2_ref[...]
    o_ref[...] = jax.nn.sigmoid(logits)


def _mlp(u_emb, i_emb, W1, b1, W2, b2, B, blk=2048):
    grid = B // blk
    return pl.pallas_call(
        _mlp_body,
        grid=(grid,),
        in_specs=[
            pl.BlockSpec((blk, D), lambda i: (i, 0)),
            pl.BlockSpec((blk, D), lambda i: (i, 0)),
            pl.BlockSpec((3 * D, 8), lambda i: (0, 0)),
            pl.BlockSpec((1, 8), lambda i: (0, 0)),
            pl.BlockSpec((8, 1), lambda i: (0, 0)),
            pl.BlockSpec((1, 1), lambda i: (0, 0)),
        ],
        out_specs=pl.BlockSpec((blk, 1), lambda i: (i, 0)),
        out_shape=jax.ShapeDtypeStruct((B, 1), jnp.float32),
        compiler_params=pltpu.CompilerParams(
            dimension_semantics=("arbitrary",)),
    )(u_emb, i_emb, W1, b1, W2, b2)


def kernel(user_inputs, item_inputs, user_table, item_table, W1, b1, W2, b2):
    B = user_inputs.shape[0]
    sc_gather, NW, n_chunks = _make_sc_gather(
        user_table.shape[0], item_table.shape[0], B)
    uidx = user_inputs.reshape(NW, n_chunks, IDX_CHUNK)
    iidx = item_inputs.reshape(NW, n_chunks, IDX_CHUNK)
    u_emb, i_emb = sc_gather(uidx, iidx, user_table, item_table)
    u_emb = u_emb.reshape(B, D)
    i_emb = i_emb.reshape(B, D)
    return _mlp(u_emb, i_emb, W1, b1.reshape(1, 8), W2, b2.reshape(1, 1), B)


# speedup vs baseline: 1.6376x; 1.6376x over previous
import functools

import jax
import jax.numpy as jnp
from jax import lax
from jax.experimental import pallas as pl
from jax.experimental.pallas import tpu as pltpu
from jax.experimental.pallas import tpu_sc as plsc

D = 64
CH = 256


def _make_sc_gather(B):
    info = plsc.get_sparse_core_info()
    NC, NS = info.num_cores, info.num_subcores
    NW = NC * NS
    assert B % NW == 0
    bpw = B // NW

    mesh = plsc.VectorSubcoreMesh(core_axis_name="c", subcore_axis_name="s")

    @functools.partial(
        pl.kernel,
        mesh=mesh,
        out_type=(
            jax.ShapeDtypeStruct((B, D), jnp.float32),
            jax.ShapeDtypeStruct((B, D), jnp.float32),
        ),
        scratch_types=[
            pltpu.VMEM((bpw,), jnp.int32),
            pltpu.VMEM((bpw,), jnp.int32),
            pltpu.VMEM((CH, D), jnp.float32),
            pltpu.VMEM((CH, D), jnp.float32),
            pltpu.SemaphoreType.DMA,
        ],
    )
    def sc_gather(uidx_hbm, iidx_hbm, utab_hbm, itab_hbm, uout_hbm, iout_hbm,
                  uidx_v, iidx_v, urows_v, irows_v, sem):
        wid = lax.axis_index("s") * NC + lax.axis_index("c")
        base = wid * bpw
        pltpu.sync_copy(uidx_hbm.at[pl.ds(base, bpw)], uidx_v)
        pltpu.sync_copy(iidx_hbm.at[pl.ds(base, bpw)], iidx_v)

        for c in range(bpw // CH):
            off = c * CH

            def body(g, _):
                uvec = uidx_v[pl.ds(off + g * 16, 16)]
                ivec = iidx_v[pl.ds(off + g * 16, 16)]
                for j in range(16):
                    u = uvec[j]
                    t = ivec[j]
                    pltpu.async_copy(utab_hbm.at[pl.ds(u, 1)],
                                     urows_v.at[pl.ds(g * 16 + j, 1)], sem)
                    pltpu.async_copy(itab_hbm.at[pl.ds(t, 1)],
                                     irows_v.at[pl.ds(g * 16 + j, 1)], sem)
                return _

            lax.fori_loop(0, CH // 16, body, None)
            pltpu.make_async_copy(utab_hbm.at[pl.ds(0, CH)], urows_v,
                                  sem).wait()
            pltpu.make_async_copy(itab_hbm.at[pl.ds(0, CH)], irows_v,
                                  sem).wait()
            pltpu.sync_copy(urows_v, uout_hbm.at[pl.ds(base + off, CH)])
            pltpu.sync_copy(irows_v, iout_hbm.at[pl.ds(base + off, CH)])

    return sc_gather


def _mlp_body(u_ref, i_ref, w1_ref, b1_ref, w2_ref, b2_ref, o_ref):
    u = u_ref[...]
    it = i_ref[...]
    e = u * it
    x = jnp.concatenate([e, u, it], axis=1)
    h = jnp.maximum(
        jnp.dot(x, w1_ref[...], preferred_element_type=jnp.float32)
        + b1_ref[...], 0.0)
    logits = jnp.dot(h, w2_ref[...], preferred_element_type=jnp.float32) \
        + b2_ref[...]
    o_ref[...] = jax.nn.sigmoid(logits)


def _mlp(u_emb, i_emb, W1, b1, W2, b2, B, blk=2048):
    grid = B // blk
    return pl.pallas_call(
        _mlp_body,
        grid=(grid,),
        in_specs=[
            pl.BlockSpec((blk, D), lambda i: (i, 0)),
            pl.BlockSpec((blk, D), lambda i: (i, 0)),
            pl.BlockSpec((3 * D, 8), lambda i: (0, 0)),
            pl.BlockSpec((1, 8), lambda i: (0, 0)),
            pl.BlockSpec((8, 1), lambda i: (0, 0)),
            pl.BlockSpec((1, 1), lambda i: (0, 0)),
        ],
        out_specs=pl.BlockSpec((blk, 1), lambda i: (i, 0)),
        out_shape=jax.ShapeDtypeStruct((B, 1), jnp.float32),
        compiler_params=pltpu.CompilerParams(
            dimension_semantics=("arbitrary",)),
    )(u_emb, i_emb, W1, b1, W2, b2)


def kernel(user_inputs, item_inputs, user_table, item_table, W1, b1, W2, b2):
    B = user_inputs.shape[0]
    sc_gather = _make_sc_gather(B)
    u_emb, i_emb = sc_gather(user_inputs, item_inputs, user_table, item_table)
    return _mlp(u_emb, i_emb, W1, b1.reshape(1, 8), W2, b2.reshape(1, 1), B)


# speedup vs baseline: 1.6440x; 1.0039x over previous
import functools

import jax
import jax.numpy as jnp
from jax import lax
from jax.experimental import pallas as pl
from jax.experimental.pallas import tpu as pltpu
from jax.experimental.pallas import tpu_sc as plsc

D = 64
CH = 256


def _tp_body(tabT_ref, o_ref):
    o_ref[...] = tabT_ref[...].T


def _transpose(tabT, bk=4096):
    n = tabT.shape[1]
    grid = (n + bk - 1) // bk
    return pl.pallas_call(
        _tp_body,
        grid=(grid,),
        in_specs=[pl.BlockSpec((D, bk), lambda i: (0, i))],
        out_specs=pl.BlockSpec((bk, D), lambda i: (i, 0)),
        out_shape=jax.ShapeDtypeStruct((n, D), jnp.float32),
        compiler_params=pltpu.CompilerParams(
            dimension_semantics=("arbitrary",)),
    )(tabT)


def _make_sc_gather(B):
    info = plsc.get_sparse_core_info()
    NC, NS = info.num_cores, info.num_subcores
    NW = NC * NS
    assert B % NW == 0
    bpw = B // NW

    mesh = plsc.VectorSubcoreMesh(core_axis_name="c", subcore_axis_name="s")

    @functools.partial(
        pl.kernel,
        mesh=mesh,
        out_type=(
            jax.ShapeDtypeStruct((B, D), jnp.float32),
            jax.ShapeDtypeStruct((B, D), jnp.float32),
        ),
        scratch_types=[
            pltpu.VMEM((bpw,), jnp.int32),
            pltpu.VMEM((bpw,), jnp.int32),
            pltpu.VMEM((CH, D), jnp.float32),
            pltpu.VMEM((CH, D), jnp.float32),
            pltpu.SemaphoreType.DMA,
        ],
    )
    def sc_gather(uidx_hbm, iidx_hbm, utab_hbm, itab_hbm, uout_hbm, iout_hbm,
                  uidx_v, iidx_v, urows_v, irows_v, sem):
        wid = lax.axis_index("s") * NC + lax.axis_index("c")
        base = wid * bpw
        pltpu.sync_copy(uidx_hbm.at[pl.ds(base, bpw)], uidx_v)
        pltpu.sync_copy(iidx_hbm.at[pl.ds(base, bpw)], iidx_v)

        for c in range(bpw // CH):
            off = c * CH

            def body(g, _):
                uvec = uidx_v[pl.ds(off + g * 16, 16)]
                ivec = iidx_v[pl.ds(off + g * 16, 16)]
                for j in range(16):
                    u = uvec[j]
                    t = ivec[j]
                    i = g * 16 + j
                    pltpu.async_copy(utab_hbm.at[pl.ds(u, 1)],
                                     urows_v.at[pl.ds(i, 1)], sem)
                    pltpu.async_copy(itab_hbm.at[pl.ds(t, 1)],
                                     irows_v.at[pl.ds(i, 1)], sem)
                return _

            lax.fori_loop(0, CH // 16, body, None)
            pltpu.make_async_copy(utab_hbm.at[pl.ds(0, CH)], urows_v,
                                  sem).wait()
            pltpu.make_async_copy(itab_hbm.at[pl.ds(0, CH)], irows_v,
                                  sem).wait()
            pltpu.sync_copy(urows_v, uout_hbm.at[pl.ds(base + off, CH)])
            pltpu.sync_copy(irows_v, iout_hbm.at[pl.ds(base + off, CH)])

    return sc_gather


def _mlp_body(u_ref, i_ref, w1_ref, b1_ref, w2_ref, b2_ref, o_ref):
    u = u_ref[...]
    it = i_ref[...]
    e = u * it
    x = jnp.concatenate([e, u, it], axis=1)
    h = jnp.maximum(
        jnp.dot(x, w1_ref[...], preferred_element_type=jnp.float32)
        + b1_ref[...], 0.0)
    logits = jnp.dot(h, w2_ref[...], preferred_element_type=jnp.float32) \
        + b2_ref[...]
    o_ref[...] = jax.nn.sigmoid(logits)


def _mlp(u_emb, i_emb, W1, b1, W2, b2, B, blk=2048):
    grid = B // blk
    return pl.pallas_call(
        _mlp_body,
        grid=(grid,),
        in_specs=[
            pl.BlockSpec((blk, D), lambda i: (i, 0)),
            pl.BlockSpec((blk, D), lambda i: (i, 0)),
            pl.BlockSpec((3 * D, 8), lambda i: (0, 0)),
            pl.BlockSpec((1, 8), lambda i: (0, 0)),
            pl.BlockSpec((8, 1), lambda i: (0, 0)),
            pl.BlockSpec((1, 1), lambda i: (0, 0)),
        ],
        out_specs=pl.BlockSpec((blk, 1), lambda i: (i, 0)),
        out_shape=jax.ShapeDtypeStruct((B, 1), jnp.float32),
        compiler_params=pltpu.CompilerParams(
            dimension_semantics=("arbitrary",)),
    )(u_emb, i_emb, W1, b1, W2, b2)


def kernel(user_inputs, item_inputs, user_table, item_table, W1, b1, W2, b2):
    B = user_inputs.shape[0]
    utab_t = _transpose(user_table.T)
    itab_t = _transpose(item_table.T)
    sc_gather = _make_sc_gather(B)
    u_emb, i_emb = sc_gather(user_inputs, item_inputs, utab_t, itab_t)
    return _mlp(u_emb, i_emb, W1, b1.reshape(1, 8), W2, b2.reshape(1, 1), B)
